# speedup vs baseline: 2.9142x; 2.9142x over previous
import functools

import jax
import jax.numpy as jnp
from jax import lax
from jax.experimental import pallas as pl
from jax.experimental.pallas import tpu as pltpu
from jax.experimental.pallas import tpu_sc as plsc

_NUM_CORES = 2
_NUM_SUBCORES = 16
_NUM_WORKERS = _NUM_CORES * _NUM_SUBCORES
_CHUNK = 32
_NBUF = 2


@functools.lru_cache(maxsize=None)
def _build_copy_kernel(B, L, D, dtype):
    rows_per_w = L // _NUM_WORKERS
    n_chunks = rows_per_w // _CHUNK
    mesh = plsc.VectorSubcoreMesh(core_axis_name="c", subcore_axis_name="s")

    @functools.partial(
        pl.kernel,
        mesh=mesh,
        out_type=jax.ShapeDtypeStruct((B * L, D), dtype),
        scratch_types=[
            pltpu.VMEM((_NBUF, _CHUNK, D), dtype),
            pltpu.SemaphoreType.DMA,
            pltpu.SemaphoreType.DMA,
        ],
    )
    def copy_kernel(w_hbm, out_hbm, buf, in_sem, out_sem):
        wid = lax.axis_index("s") * _NUM_CORES + lax.axis_index("c")
        base = wid * rows_per_w

        def start_read(c):
            return pltpu.async_copy(
                w_hbm.at[pl.ds(base + c * _CHUNK, _CHUNK)],
                buf.at[c % _NBUF],
                in_sem,
            )

        def start_writes(c):
            return [
                pltpu.async_copy(
                    buf.at[c % _NBUF],
                    out_hbm.at[pl.ds(b * L + base + c * _CHUNK, _CHUNK)],
                    out_sem,
                )
                for b in range(B)
            ]

        reads = {0: start_read(0)}
        pending_writes = []
        for c in range(n_chunks):
            reads.pop(c).wait()
            if c + 1 < n_chunks:
                if len(pending_writes) >= _NBUF - 1:
                    for w in pending_writes.pop(0):
                        w.wait()
                reads[c + 1] = start_read(c + 1)
            pending_writes.append(start_writes(c))
        for group in pending_writes:
            for w in group:
                w.wait()

    return copy_kernel


def kernel(x, W):
    B, L, D = x.shape
    out_flat = _build_copy_kernel(B, L, D, W.dtype)(W[:L])
    return out_flat.reshape(B, L, D)
